# baseline (speedup 1.0000x reference)
import functools

import jax
import jax.numpy as jnp
from jax import lax
from jax.experimental import pallas as pl
from jax.experimental.pallas import tpu as pltpu
from jax.experimental.pallas import tpu_sc as plsc

MAXLEN = 200
EMBED_DIM = 32
NUM_WORKERS = 32
SEQ_PER_CHUNK = 8
CHUNK = SEQ_PER_CHUNK * MAXLEN


def _gather_slices(chunk):
    slices = []
    off = 0
    while off < chunk:
        size = min(128, chunk - off)
        slices.append((off, size))
        off += size
    return slices


TBLK = 1024
N_TBLK_FULL = 976
TAIL_OFF = N_TBLK_FULL * TBLK
TAIL_COLS = 512
TAIL64_OFF = TAIL_OFF + TAIL_COLS
VOCAB_PAD = 1000064


def _transpose_table(ttT, tail64):
    mesh = plsc.VectorSubcoreMesh(core_axis_name="c", subcore_axis_name="s")

    @functools.partial(
        pl.kernel,
        mesh=mesh,
        out_type=jax.ShapeDtypeStruct((VOCAB_PAD // 4, 128), jnp.float32),
        scratch_types=[
            pltpu.VMEM((EMBED_DIM, TBLK), jnp.float32),
            pltpu.VMEM((TBLK // 4, 128), jnp.float32),
            pltpu.VMEM((EMBED_DIM, 64), jnp.float32),
        ],
        compiler_params=pltpu.CompilerParams(
            use_tc_tiling_on_sc=True, needs_layout_passes=False
        ),
    )
    def tk(in_hbm, tail_hbm, out_hbm, in_v, out_v, tail_v):
        wid = lax.axis_index("s") * 2 + lax.axis_index("c")
        lanes = jnp.arange(16, dtype=jnp.int32)

        def transpose_rows(src_v, n_rows):
            def row_body(j, carry):
                jv = jnp.full((16,), j, dtype=jnp.int32)
                for q in range(4):
                    m = jnp.full((16,), j * 4 + q, dtype=jnp.int32)
                    for h in range(2):
                        v = plsc.load_gather(src_v, [lanes + h * 16, m])
                        plsc.store_scatter(out_v, [jv, lanes + (q * 32 + h * 16)], v)
                return carry

            lax.fori_loop(0, n_rows, row_body, 0)

        def do_block(c0, n_cols):
            c0 = pl.multiple_of(c0, 512)
            pltpu.sync_copy(in_hbm.at[:, pl.ds(c0, n_cols)], in_v.at[:, pl.ds(0, n_cols)])
            transpose_rows(in_v, n_cols // 4)
            pltpu.sync_copy(
                out_v.at[pl.ds(0, n_cols // 4)],
                out_hbm.at[pl.ds(pl.multiple_of(c0 // 4, 128), n_cols // 4)],
            )

        def blk_body(i, carry):
            b = i * NUM_WORKERS + wid

            @pl.when(b < N_TBLK_FULL)
            def _():
                do_block(b * TBLK, TBLK)

            @pl.when(b == N_TBLK_FULL)
            def _():
                do_block(TAIL_OFF, TAIL_COLS)

            return carry

        n_iter = (N_TBLK_FULL + 1 + NUM_WORKERS - 1) // NUM_WORKERS
        lax.fori_loop(0, n_iter, blk_body, 0)

        @pl.when(wid == NUM_WORKERS - 1)
        def _():
            pltpu.sync_copy(tail_hbm, tail_v)
            transpose_rows(tail_v, 16)
            pltpu.sync_copy(
                out_v.at[pl.ds(0, 16)],
                out_hbm.at[pl.ds(TAIL64_OFF // 4, 16)],
            )

    return tk(ttT, tail64)


@functools.partial(jax.jit, static_argnames=("n_rows",))
def _embed(x_flat, token_table, pos_table, n_rows):
    per_w = n_rows // NUM_WORKERS
    n_chunks = per_w // CHUNK
    slices = _gather_slices(CHUNK)
    mesh = plsc.VectorSubcoreMesh(core_axis_name="c", subcore_axis_name="s")

    @functools.partial(
        pl.kernel,
        mesh=mesh,
        out_type=jax.ShapeDtypeStruct((n_rows, EMBED_DIM), jnp.float32),
        scratch_types=[
            pltpu.VMEM((CHUNK,), jnp.int32),
            pltpu.VMEM((CHUNK, EMBED_DIM), jnp.float32),
            pltpu.VMEM((MAXLEN, EMBED_DIM), jnp.float32),
            pltpu.SemaphoreType.DMA,
        ],
        compiler_params=pltpu.CompilerParams(use_tc_tiling_on_sc=False),
    )
    def k(x_hbm, tok_hbm, pos_hbm, out_hbm, idx_v, rows_v, pos_v, sem):
        wid = lax.axis_index("s") * 2 + lax.axis_index("c")
        base = wid * per_w
        pltpu.sync_copy(pos_hbm, pos_v)

        def chunk_body(c, carry):
            off = base + c * CHUNK
            pltpu.sync_copy(x_hbm.at[pl.ds(off, CHUNK)], idx_v)
            for s_off, s_size in slices:
                pltpu.async_copy(
                    tok_hbm.at[idx_v.at[pl.ds(s_off, s_size)]],
                    rows_v.at[pl.ds(s_off, s_size)],
                    sem,
                )
            for s_off, s_size in slices:
                pltpu.make_async_copy(
                    tok_hbm.at[idx_v.at[pl.ds(s_off, s_size)]],
                    rows_v.at[pl.ds(s_off, s_size)],
                    sem,
                ).wait()

            def seq_body(s, carry2):
                def row_body(p, carry3):
                    r = s * MAXLEN + p
                    rows_v[r, pl.ds(0, 16)] = (
                        rows_v[r, pl.ds(0, 16)] + pos_v[p, pl.ds(0, 16)]
                    )
                    rows_v[r, pl.ds(16, 16)] = (
                        rows_v[r, pl.ds(16, 16)] + pos_v[p, pl.ds(16, 16)]
                    )
                    return carry3

                return lax.fori_loop(0, MAXLEN, row_body, carry2)

            lax.fori_loop(0, SEQ_PER_CHUNK, seq_body, 0)
            pltpu.sync_copy(rows_v, out_hbm.at[pl.ds(off, CHUNK)])
            return carry

        lax.fori_loop(0, n_chunks, chunk_body, 0)

    return k(x_flat, token_table, pos_table)


def kernel(x, token_table, pos_table):
    batch, maxlen = x.shape
    vocab = token_table.shape[0]
    n_rows = batch * maxlen
    x_flat = x.reshape(n_rows).astype(jnp.int32)
    ttT = token_table.T
    t4 = _transpose_table(ttT, ttT[:, TAIL64_OFF:])
    table_lin = t4.reshape(VOCAB_PAD, EMBED_DIM)
    out = _embed(x_flat, table_lin, pos_table, n_rows)
    return out.reshape(batch, maxlen, EMBED_DIM)


# speedup vs baseline: 1.0007x; 1.0007x over previous
import functools

import jax
import jax.numpy as jnp
from jax import lax
from jax.experimental import pallas as pl
from jax.experimental.pallas import tpu as pltpu
from jax.experimental.pallas import tpu_sc as plsc

MAXLEN = 200
EMBED_DIM = 32
NUM_WORKERS = 32
SEQ_PER_CHUNK = 8
CHUNK = SEQ_PER_CHUNK * MAXLEN


def _gather_slices(chunk):
    slices = []
    off = 0
    while off < chunk:
        size = min(128, chunk - off)
        slices.append((off, size))
        off += size
    return slices


TBLK = 1024
N_TBLK_FULL = 976
TAIL_OFF = N_TBLK_FULL * TBLK
TAIL_COLS = 512
TAIL64_OFF = TAIL_OFF + TAIL_COLS
VOCAB_PAD = 1000064


def _transpose_table(ttT, tail64):
    mesh = plsc.VectorSubcoreMesh(core_axis_name="c", subcore_axis_name="s")

    @functools.partial(
        pl.kernel,
        mesh=mesh,
        out_type=jax.ShapeDtypeStruct((VOCAB_PAD // 4, 128), jnp.float32),
        scratch_types=[
            pltpu.VMEM((EMBED_DIM, TBLK), jnp.float32),
            pltpu.VMEM((TBLK // 4, 128), jnp.float32),
            pltpu.VMEM((EMBED_DIM, 64), jnp.float32),
        ],
        compiler_params=pltpu.CompilerParams(
            use_tc_tiling_on_sc=True, needs_layout_passes=False
        ),
    )
    def tk(in_hbm, tail_hbm, out_hbm, in_v, out_v, tail_v):
        wid = lax.axis_index("s") * 2 + lax.axis_index("c")
        lanes = jnp.arange(16, dtype=jnp.int32)

        def transpose_rows(src_v, n_rows):
            def row_body(j, carry):
                for q in range(4):
                    m = jnp.full((16,), j * 4 + q, dtype=jnp.int32)
                    for h in range(2):
                        v = plsc.load_gather(src_v, [lanes + h * 16, m])
                        out_v[j, pl.ds(q * 32 + h * 16, 16)] = v
                return carry

            lax.fori_loop(0, n_rows, row_body, 0)

        def do_block(c0, n_cols):
            c0 = pl.multiple_of(c0, 512)
            pltpu.sync_copy(in_hbm.at[:, pl.ds(c0, n_cols)], in_v.at[:, pl.ds(0, n_cols)])
            transpose_rows(in_v, n_cols // 4)
            pltpu.sync_copy(
                out_v.at[pl.ds(0, n_cols // 4)],
                out_hbm.at[pl.ds(pl.multiple_of(c0 // 4, 128), n_cols // 4)],
            )

        def blk_body(i, carry):
            b = i * NUM_WORKERS + wid

            @pl.when(b < N_TBLK_FULL)
            def _():
                do_block(b * TBLK, TBLK)

            @pl.when(b == N_TBLK_FULL)
            def _():
                do_block(TAIL_OFF, TAIL_COLS)

            return carry

        n_iter = (N_TBLK_FULL + 1 + NUM_WORKERS - 1) // NUM_WORKERS
        lax.fori_loop(0, n_iter, blk_body, 0)

        @pl.when(wid == NUM_WORKERS - 1)
        def _():
            pltpu.sync_copy(tail_hbm, tail_v)
            transpose_rows(tail_v, 16)
            pltpu.sync_copy(
                out_v.at[pl.ds(0, 16)],
                out_hbm.at[pl.ds(TAIL64_OFF // 4, 16)],
            )

    return tk(ttT, tail64)


@functools.partial(jax.jit, static_argnames=("n_rows",))
def _embed(x_flat, token_table, pos_table, n_rows):
    per_w = n_rows // NUM_WORKERS
    n_chunks = per_w // CHUNK
    slices = _gather_slices(CHUNK)
    mesh = plsc.VectorSubcoreMesh(core_axis_name="c", subcore_axis_name="s")

    @functools.partial(
        pl.kernel,
        mesh=mesh,
        out_type=jax.ShapeDtypeStruct((n_rows, EMBED_DIM), jnp.float32),
        scratch_types=[
            pltpu.VMEM((CHUNK,), jnp.int32),
            pltpu.VMEM((CHUNK, EMBED_DIM), jnp.float32),
            pltpu.VMEM((MAXLEN, EMBED_DIM), jnp.float32),
            pltpu.SemaphoreType.DMA,
        ],
        compiler_params=pltpu.CompilerParams(use_tc_tiling_on_sc=False),
    )
    def k(x_hbm, tok_hbm, pos_hbm, out_hbm, idx_v, rows_v, pos_v, sem):
        wid = lax.axis_index("s") * 2 + lax.axis_index("c")
        base = wid * per_w
        pltpu.sync_copy(pos_hbm, pos_v)

        def chunk_body(c, carry):
            off = base + c * CHUNK
            pltpu.sync_copy(x_hbm.at[pl.ds(off, CHUNK)], idx_v)
            for s_off, s_size in slices:
                pltpu.async_copy(
                    tok_hbm.at[idx_v.at[pl.ds(s_off, s_size)]],
                    rows_v.at[pl.ds(s_off, s_size)],
                    sem,
                )
            for s_off, s_size in slices:
                pltpu.make_async_copy(
                    tok_hbm.at[idx_v.at[pl.ds(s_off, s_size)]],
                    rows_v.at[pl.ds(s_off, s_size)],
                    sem,
                ).wait()

            def seq_body(s, carry2):
                def row_body(p, carry3):
                    r = s * MAXLEN + p
                    rows_v[r, pl.ds(0, 16)] = (
                        rows_v[r, pl.ds(0, 16)] + pos_v[p, pl.ds(0, 16)]
                    )
                    rows_v[r, pl.ds(16, 16)] = (
                        rows_v[r, pl.ds(16, 16)] + pos_v[p, pl.ds(16, 16)]
                    )
                    return carry3

                return lax.fori_loop(0, MAXLEN, row_body, carry2)

            lax.fori_loop(0, SEQ_PER_CHUNK, seq_body, 0)
            pltpu.sync_copy(rows_v, out_hbm.at[pl.ds(off, CHUNK)])
            return carry

        lax.fori_loop(0, n_chunks, chunk_body, 0)

    return k(x_flat, token_table, pos_table)


def kernel(x, token_table, pos_table):
    batch, maxlen = x.shape
    vocab = token_table.shape[0]
    n_rows = batch * maxlen
    x_flat = x.reshape(n_rows).astype(jnp.int32)
    ttT = token_table.T
    t4 = _transpose_table(ttT, ttT[:, TAIL64_OFF:])
    table_lin = t4.reshape(VOCAB_PAD, EMBED_DIM)
    out = _embed(x_flat, table_lin, pos_table, n_rows)
    return out.reshape(batch, maxlen, EMBED_DIM)


# speedup vs baseline: 1.3422x; 1.3413x over previous
import functools

import jax
import jax.numpy as jnp
from jax import lax
from jax.experimental import pallas as pl
from jax.experimental.pallas import tpu as pltpu
from jax.experimental.pallas import tpu_sc as plsc

MAXLEN = 200
EMBED_DIM = 32
NUM_WORKERS = 32
SEQ_PER_CHUNK = 8
CHUNK = SEQ_PER_CHUNK * MAXLEN


def _gather_slices(chunk):
    slices = []
    off = 0
    while off < chunk:
        size = min(128, chunk - off)
        slices.append((off, size))
        off += size
    return slices


@functools.partial(jax.jit, static_argnames=("n_rows",))
def _embed(x_flat, token_table, pos_table, n_rows):
    per_w = n_rows // NUM_WORKERS
    n_chunks = per_w // CHUNK
    slices = _gather_slices(CHUNK)
    mesh = plsc.VectorSubcoreMesh(core_axis_name="c", subcore_axis_name="s")

    @functools.partial(
        pl.kernel,
        mesh=mesh,
        out_type=jax.ShapeDtypeStruct((n_rows, EMBED_DIM), jnp.float32),
        scratch_types=[
            pltpu.VMEM((CHUNK,), jnp.int32),
            pltpu.VMEM((CHUNK, EMBED_DIM), jnp.float32),
            pltpu.VMEM((MAXLEN, EMBED_DIM), jnp.float32),
            pltpu.SemaphoreType.DMA,
        ],
        compiler_params=pltpu.CompilerParams(use_tc_tiling_on_sc=False),
    )
    def k(x_hbm, tok_hbm, pos_hbm, out_hbm, idx_v, rows_v, pos_v, sem):
        wid = lax.axis_index("s") * 2 + lax.axis_index("c")
        base = wid * per_w
        pltpu.sync_copy(pos_hbm, pos_v)

        def chunk_body(c, carry):
            off = base + c * CHUNK
            pltpu.sync_copy(x_hbm.at[pl.ds(off, CHUNK)], idx_v)
            for s_off, s_size in slices:
                pltpu.async_copy(
                    tok_hbm.at[idx_v.at[pl.ds(s_off, s_size)]],
                    rows_v.at[pl.ds(s_off, s_size)],
                    sem,
                )
            for s_off, s_size in slices:
                pltpu.make_async_copy(
                    tok_hbm.at[idx_v.at[pl.ds(s_off, s_size)]],
                    rows_v.at[pl.ds(s_off, s_size)],
                    sem,
                ).wait()

            def seq_body(s, carry2):
                def row_body(p, carry3):
                    r = s * MAXLEN + p
                    rows_v[r, pl.ds(0, 16)] = (
                        rows_v[r, pl.ds(0, 16)] + pos_v[p, pl.ds(0, 16)]
                    )
                    rows_v[r, pl.ds(16, 16)] = (
                        rows_v[r, pl.ds(16, 16)] + pos_v[p, pl.ds(16, 16)]
                    )
                    return carry3

                return lax.fori_loop(0, MAXLEN, row_body, carry2)

            lax.fori_loop(0, SEQ_PER_CHUNK, seq_body, 0)
            pltpu.sync_copy(rows_v, out_hbm.at[pl.ds(off, CHUNK)])
            return carry

        lax.fori_loop(0, n_chunks, chunk_body, 0)

    return k(x_flat, token_table, pos_table)


def kernel(x, token_table, pos_table):
    batch, maxlen = x.shape
    n_rows = batch * maxlen
    x_flat = x.reshape(n_rows).astype(jnp.int32)
    out = _embed(x_flat, token_table, pos_table, n_rows)
    return out.reshape(batch, maxlen, EMBED_DIM)
